# baseline (speedup 1.0000x reference)
import jax
import jax.numpy as jnp
from jax.experimental import pallas as pl
from jax.experimental.pallas import tpu as pltpu
from jax.experimental.pallas import tpu_sc as plsc

_WINDOW = 256
_K1_COLS = 2048
_K3_B = 512


def _pack_pairs_body(lo_ref, hi_ref, o_ref):
    o_ref[...] = jnp.concatenate([lo_ref[...].T, hi_ref[...].T], axis=1)


def _select_transpose_body(x_ref, p_ref, o_ref):
    x = x_ref[0]
    p = p_ref[0]
    dim = x.shape[1] // 2
    xsel = jnp.where(p.T == 1, x[:, dim:], x[:, :dim])
    o_ref[0] = xsel.T


def kernel(input_ids, table):
    batch, fields = input_ids.shape
    emb, dim = table.shape
    num_idx = batch * fields
    assert num_idx % _WINDOW == 0
    grid = num_idx // _WINDOW

    half = 524288
    assert emb <= 2 * half and half % _K1_COLS == 0
    hi_blocks = half // _K1_COLS
    max_block = (emb + _K1_COLS - 1) // _K1_COLS - 1
    table_t = table.T
    k1 = pl.pallas_call(
        _pack_pairs_body,
        grid=(hi_blocks,),
        in_specs=[
            pl.BlockSpec((dim, _K1_COLS), lambda i: (0, i)),
            pl.BlockSpec(
                (dim, _K1_COLS),
                lambda i: (0, jnp.minimum(i + hi_blocks, max_block)),
            ),
        ],
        out_specs=pl.BlockSpec((_K1_COLS, 2 * dim), lambda i: (i, 0)),
        out_shape=jax.ShapeDtypeStruct((half, 2 * dim), table.dtype),
    )
    table2 = k1(table_t, table_t)

    idx_fm = input_ids.T.reshape(grid, 1, _WINDOW)
    idx_pair = idx_fm & (half - 1)

    mesh = plsc.VectorSubcoreMesh(
        core_axis_name="core", subcore_axis_name="subcore"
    )

    @pl.kernel(
        out_type=jax.ShapeDtypeStruct((num_idx, 2 * dim), table.dtype),
        mesh=mesh,
    )
    def gather_kernel(table_hbm, idx_hbm, out_hbm):
        def body(idx_vmem, out_vmem):
            pltpu.sync_copy(table_hbm.at[idx_vmem.at[0, 0]], out_vmem)

        pltpu.emit_pipeline(
            body,
            grid=(grid,),
            in_specs=[
                pl.BlockSpec((1, 1, _WINDOW), index_map=lambda i: (i, 0, 0))
            ],
            out_specs=[
                pl.BlockSpec((_WINDOW, 2 * dim), index_map=lambda i: (i, 0))
            ],
            core_axis_name=("core", "subcore"),
            dimension_semantics=(pltpu.PARALLEL,),
        )(idx_hbm, out_hbm)

    pairs = gather_kernel(table2, idx_pair)

    pairs3 = pairs.reshape(fields, batch, 2 * dim)
    parity3 = (input_ids.T >> 19).reshape(fields, 1, batch)
    k3 = pl.pallas_call(
        _select_transpose_body,
        grid=(fields, batch // _K3_B),
        in_specs=[
            pl.BlockSpec((1, _K3_B, 2 * dim), lambda f, j: (f, j, 0)),
            pl.BlockSpec((1, 1, _K3_B), lambda f, j: (f, 0, j)),
        ],
        out_specs=pl.BlockSpec((1, dim, _K3_B), lambda f, j: (f, 0, j)),
        out_shape=jax.ShapeDtypeStruct((fields, dim, batch), table.dtype),
    )
    out_t = k3(pairs3, parity3)

    return jnp.transpose(out_t, (2, 0, 1))


# speedup vs baseline: 1.8759x; 1.8759x over previous
import jax
import jax.numpy as jnp
from jax.experimental import pallas as pl
from jax.experimental.pallas import tpu as pltpu
from jax.experimental.pallas import tpu_sc as plsc

_WINDOW = 256
_K1_COLS = 8192
_K3_B = 2048


def _pack_pairs_body(lo_ref, hi_ref, o_ref):
    o_ref[...] = jnp.concatenate([lo_ref[...].T, hi_ref[...].T], axis=1)


def _select_transpose_body(x_ref, p_ref, o_ref):
    x = x_ref[0]
    p = p_ref[0]
    xt = x.T
    dim = xt.shape[0] // 2
    o_ref[0] = jnp.where(p == 1, xt[dim:, :], xt[:dim, :])


def kernel(input_ids, table):
    batch, fields = input_ids.shape
    emb, dim = table.shape
    num_idx = batch * fields
    assert num_idx % _WINDOW == 0
    grid = num_idx // _WINDOW

    half = 524288
    assert emb <= 2 * half and half % _K1_COLS == 0
    hi_blocks = half // _K1_COLS
    max_block = (emb + _K1_COLS - 1) // _K1_COLS - 1
    table_t = table.T
    k1 = pl.pallas_call(
        _pack_pairs_body,
        grid=(hi_blocks,),
        in_specs=[
            pl.BlockSpec((dim, _K1_COLS), lambda i: (0, i)),
            pl.BlockSpec(
                (dim, _K1_COLS),
                lambda i: (0, jnp.minimum(i + hi_blocks, max_block)),
            ),
        ],
        out_specs=pl.BlockSpec((_K1_COLS, 2 * dim), lambda i: (i, 0)),
        out_shape=jax.ShapeDtypeStruct((half, 2 * dim), table.dtype),
    )
    table2 = k1(table_t, table_t)

    idx_fm = input_ids.T.reshape(grid, 1, _WINDOW)
    idx_pair = idx_fm & (half - 1)

    mesh = plsc.VectorSubcoreMesh(
        core_axis_name="core", subcore_axis_name="subcore"
    )

    @pl.kernel(
        out_type=jax.ShapeDtypeStruct((num_idx, 2 * dim), table.dtype),
        mesh=mesh,
    )
    def gather_kernel(table_hbm, idx_hbm, out_hbm):
        def body(idx_vmem, out_vmem):
            pltpu.sync_copy(table_hbm.at[idx_vmem.at[0, 0]], out_vmem)

        pltpu.emit_pipeline(
            body,
            grid=(grid,),
            in_specs=[
                pl.BlockSpec((1, 1, _WINDOW), index_map=lambda i: (i, 0, 0))
            ],
            out_specs=[
                pl.BlockSpec((_WINDOW, 2 * dim), index_map=lambda i: (i, 0))
            ],
            core_axis_name=("core", "subcore"),
            dimension_semantics=(pltpu.PARALLEL,),
        )(idx_hbm, out_hbm)

    pairs = gather_kernel(table2, idx_pair)

    pairs3 = pairs.reshape(fields, batch, 2 * dim)
    parity3 = (input_ids.T >> 19).reshape(fields, 1, batch)
    k3 = pl.pallas_call(
        _select_transpose_body,
        grid=(fields, batch // _K3_B),
        in_specs=[
            pl.BlockSpec((1, _K3_B, 2 * dim), lambda f, j: (f, j, 0)),
            pl.BlockSpec((1, 1, _K3_B), lambda f, j: (f, 0, j)),
        ],
        out_specs=pl.BlockSpec((1, dim, _K3_B), lambda f, j: (f, 0, j)),
        out_shape=jax.ShapeDtypeStruct((fields, dim, batch), table.dtype),
    )
    out_t = k3(pairs3, parity3)

    return jnp.transpose(out_t, (2, 0, 1))


# speedup vs baseline: 2.1142x; 1.1270x over previous
import jax
import jax.numpy as jnp
from jax.experimental import pallas as pl
from jax.experimental.pallas import tpu as pltpu
from jax.experimental.pallas import tpu_sc as plsc

_WINDOW = 256
_K1_COLS = 16384
_K3_B = 4096


def _pack_pairs_body(lo_ref, hi_ref, o_ref):
    o_ref[...] = jnp.concatenate([lo_ref[...].T, hi_ref[...].T], axis=1)


def _select_transpose_body(x_ref, p_ref, o_ref):
    x = x_ref[0]
    p = p_ref[0]
    xt = x.T
    dim = xt.shape[0] // 2
    o_ref[0] = jnp.where(p == 1, xt[dim:, :], xt[:dim, :])


def kernel(input_ids, table):
    batch, fields = input_ids.shape
    emb, dim = table.shape
    num_idx = batch * fields
    assert num_idx % _WINDOW == 0
    grid = num_idx // _WINDOW

    half = 524288
    assert emb <= 2 * half and half % _K1_COLS == 0
    hi_blocks = half // _K1_COLS
    max_block = (emb + _K1_COLS - 1) // _K1_COLS - 1
    table_t = table.T
    k1 = pl.pallas_call(
        _pack_pairs_body,
        grid=(hi_blocks,),
        in_specs=[
            pl.BlockSpec((dim, _K1_COLS), lambda i: (0, i)),
            pl.BlockSpec(
                (dim, _K1_COLS),
                lambda i: (0, jnp.minimum(i + hi_blocks, max_block)),
            ),
        ],
        out_specs=pl.BlockSpec((_K1_COLS, 2 * dim), lambda i: (i, 0)),
        out_shape=jax.ShapeDtypeStruct((half, 2 * dim), table.dtype),
    )
    table2 = k1(table_t, table_t)

    idx_fm = input_ids.T.reshape(grid, 1, _WINDOW)
    idx_pair = idx_fm & (half - 1)

    mesh = plsc.VectorSubcoreMesh(
        core_axis_name="core", subcore_axis_name="subcore"
    )

    @pl.kernel(
        out_type=jax.ShapeDtypeStruct((num_idx, 2 * dim), table.dtype),
        mesh=mesh,
    )
    def gather_kernel(table_hbm, idx_hbm, out_hbm):
        def body(idx_vmem, out_vmem):
            pltpu.sync_copy(table_hbm.at[idx_vmem.at[0, 0]], out_vmem)

        pltpu.emit_pipeline(
            body,
            grid=(grid,),
            in_specs=[
                pl.BlockSpec((1, 1, _WINDOW), index_map=lambda i: (i, 0, 0))
            ],
            out_specs=[
                pl.BlockSpec((_WINDOW, 2 * dim), index_map=lambda i: (i, 0))
            ],
            core_axis_name=("core", "subcore"),
            dimension_semantics=(pltpu.PARALLEL,),
        )(idx_hbm, out_hbm)

    pairs = gather_kernel(table2, idx_pair)

    pairs3 = pairs.reshape(fields, batch, 2 * dim)
    parity3 = (input_ids.T >> 19).reshape(fields, 1, batch)
    k3 = pl.pallas_call(
        _select_transpose_body,
        grid=(fields, batch // _K3_B),
        in_specs=[
            pl.BlockSpec((1, _K3_B, 2 * dim), lambda f, j: (f, j, 0)),
            pl.BlockSpec((1, 1, _K3_B), lambda f, j: (f, 0, j)),
        ],
        out_specs=pl.BlockSpec((1, dim, _K3_B), lambda f, j: (f, 0, j)),
        out_shape=jax.ShapeDtypeStruct((fields, dim, batch), table.dtype),
    )
    out_t = k3(pairs3, parity3)

    return jnp.transpose(out_t, (2, 0, 1))


# speedup vs baseline: 2.2375x; 1.0583x over previous
import jax
import jax.numpy as jnp
from jax.experimental import pallas as pl
from jax.experimental.pallas import tpu as pltpu
from jax.experimental.pallas import tpu_sc as plsc

_WINDOW = 256
_K1_COLS = 16384
_K3_B = 8192


def _pack_pairs_body(lo_ref, hi_ref, o_ref):
    o_ref[...] = jnp.concatenate([lo_ref[...].T, hi_ref[...].T], axis=1)


def _select_transpose_body(x_ref, p_ref, o_ref):
    x = x_ref[0]
    p = p_ref[0]
    xt = x.T
    dim = xt.shape[0] // 2
    o_ref[0] = jnp.where(p == 1, xt[dim:, :], xt[:dim, :])


def kernel(input_ids, table):
    batch, fields = input_ids.shape
    emb, dim = table.shape
    num_idx = batch * fields
    assert num_idx % _WINDOW == 0
    grid = num_idx // _WINDOW

    half = 524288
    assert emb <= 2 * half and half % _K1_COLS == 0
    hi_blocks = half // _K1_COLS
    max_block = (emb + _K1_COLS - 1) // _K1_COLS - 1
    table_t = table.T
    k1 = pl.pallas_call(
        _pack_pairs_body,
        grid=(hi_blocks,),
        in_specs=[
            pl.BlockSpec((dim, _K1_COLS), lambda i: (0, i)),
            pl.BlockSpec(
                (dim, _K1_COLS),
                lambda i: (0, jnp.minimum(i + hi_blocks, max_block)),
            ),
        ],
        out_specs=pl.BlockSpec((_K1_COLS, 2 * dim), lambda i: (i, 0)),
        out_shape=jax.ShapeDtypeStruct((half, 2 * dim), table.dtype),
    )
    table2 = k1(table_t, table_t)

    idx_fm = input_ids.T.reshape(grid, 1, _WINDOW)
    idx_pair = idx_fm & (half - 1)

    mesh = plsc.VectorSubcoreMesh(
        core_axis_name="core", subcore_axis_name="subcore"
    )

    @pl.kernel(
        out_type=jax.ShapeDtypeStruct((num_idx, 2 * dim), table.dtype),
        mesh=mesh,
    )
    def gather_kernel(table_hbm, idx_hbm, out_hbm):
        def body(idx_vmem, out_vmem):
            pltpu.sync_copy(table_hbm.at[idx_vmem.at[0, 0]], out_vmem)

        pltpu.emit_pipeline(
            body,
            grid=(grid,),
            in_specs=[
                pl.BlockSpec((1, 1, _WINDOW), index_map=lambda i: (i, 0, 0))
            ],
            out_specs=[
                pl.BlockSpec((_WINDOW, 2 * dim), index_map=lambda i: (i, 0))
            ],
            core_axis_name=("core", "subcore"),
            dimension_semantics=(pltpu.PARALLEL,),
        )(idx_hbm, out_hbm)

    pairs = gather_kernel(table2, idx_pair)

    pairs3 = pairs.reshape(fields, batch, 2 * dim)
    parity3 = (input_ids.T >> 19).reshape(fields, 1, batch)
    k3 = pl.pallas_call(
        _select_transpose_body,
        grid=(fields, batch // _K3_B),
        in_specs=[
            pl.BlockSpec((1, _K3_B, 2 * dim), lambda f, j: (f, j, 0)),
            pl.BlockSpec((1, 1, _K3_B), lambda f, j: (f, 0, j)),
        ],
        out_specs=pl.BlockSpec((1, dim, _K3_B), lambda f, j: (f, 0, j)),
        out_shape=jax.ShapeDtypeStruct((fields, dim, batch), table.dtype),
    )
    out_t = k3(pairs3, parity3)

    return jnp.transpose(out_t, (2, 0, 1))


# speedup vs baseline: 2.2628x; 1.0113x over previous
import jax
import jax.numpy as jnp
from jax.experimental import pallas as pl
from jax.experimental.pallas import tpu as pltpu
from jax.experimental.pallas import tpu_sc as plsc

_WINDOW = 256
_K1_COLS = 16384
_K3_B = 16384


def _pack_pairs_body(lo_ref, hi_ref, o_ref):
    o_ref[...] = jnp.concatenate([lo_ref[...].T, hi_ref[...].T], axis=1)


def _select_transpose_body(x_ref, p_ref, o_ref):
    x = x_ref[0]
    p = p_ref[0]
    xt = x.T
    dim = xt.shape[0] // 2
    o_ref[0] = jnp.where(p == 1, xt[dim:, :], xt[:dim, :])


def kernel(input_ids, table):
    batch, fields = input_ids.shape
    emb, dim = table.shape
    num_idx = batch * fields
    assert num_idx % _WINDOW == 0
    grid = num_idx // _WINDOW

    half = 524288
    assert emb <= 2 * half and half % _K1_COLS == 0
    hi_blocks = half // _K1_COLS
    max_block = (emb + _K1_COLS - 1) // _K1_COLS - 1
    table_t = table.T
    k1 = pl.pallas_call(
        _pack_pairs_body,
        grid=(hi_blocks,),
        in_specs=[
            pl.BlockSpec((dim, _K1_COLS), lambda i: (0, i)),
            pl.BlockSpec(
                (dim, _K1_COLS),
                lambda i: (0, jnp.minimum(i + hi_blocks, max_block)),
            ),
        ],
        out_specs=pl.BlockSpec((_K1_COLS, 2 * dim), lambda i: (i, 0)),
        out_shape=jax.ShapeDtypeStruct((half, 2 * dim), table.dtype),
    )
    table2 = k1(table_t, table_t)

    idx_fm = input_ids.T.reshape(grid, 1, _WINDOW)
    idx_pair = idx_fm & (half - 1)

    mesh = plsc.VectorSubcoreMesh(
        core_axis_name="core", subcore_axis_name="subcore"
    )

    @pl.kernel(
        out_type=jax.ShapeDtypeStruct((num_idx, 2 * dim), table.dtype),
        mesh=mesh,
    )
    def gather_kernel(table_hbm, idx_hbm, out_hbm):
        def body(idx_vmem, out_vmem):
            pltpu.sync_copy(table_hbm.at[idx_vmem.at[0, 0]], out_vmem)

        pltpu.emit_pipeline(
            body,
            grid=(grid,),
            in_specs=[
                pl.BlockSpec((1, 1, _WINDOW), index_map=lambda i: (i, 0, 0))
            ],
            out_specs=[
                pl.BlockSpec((_WINDOW, 2 * dim), index_map=lambda i: (i, 0))
            ],
            core_axis_name=("core", "subcore"),
            dimension_semantics=(pltpu.PARALLEL,),
        )(idx_hbm, out_hbm)

    pairs = gather_kernel(table2, idx_pair)

    pairs3 = pairs.reshape(fields, batch, 2 * dim)
    parity3 = (input_ids.T >> 19).reshape(fields, 1, batch)
    k3 = pl.pallas_call(
        _select_transpose_body,
        grid=(fields, batch // _K3_B),
        in_specs=[
            pl.BlockSpec((1, _K3_B, 2 * dim), lambda f, j: (f, j, 0)),
            pl.BlockSpec((1, 1, _K3_B), lambda f, j: (f, 0, j)),
        ],
        out_specs=pl.BlockSpec((1, dim, _K3_B), lambda f, j: (f, 0, j)),
        out_shape=jax.ShapeDtypeStruct((fields, dim, batch), table.dtype),
    )
    out_t = k3(pairs3, parity3)

    return jnp.transpose(out_t, (2, 0, 1))
